# baseline (speedup 1.0000x reference)
import functools

import jax
import jax.numpy as jnp
from jax import lax
from jax.experimental import pallas as pl
from jax.experimental.pallas import tpu as pltpu
from jax.experimental.pallas import tpu_sc as plsc

N = 10000
E = 320000
D = 128
G = 64
OUT = 10
L = 3

NC = 2
NS = 16
NW = NC * NS
EPT = E // NW
K = 80
CH = EPT // K
BLK = 80
NB = N // BLK


def _sc_segment_sum(h, src_r, dst_r):
    mesh = plsc.VectorSubcoreMesh(core_axis_name="c", subcore_axis_name="s")

    @functools.partial(
        pl.kernel,
        mesh=mesh,
        out_type=jax.ShapeDtypeStruct((NC, N, D), jnp.float32),
        scratch_types=[
            pltpu.VMEM((EPT,), jnp.int32),
            pltpu.VMEM((CH, K), jnp.int32),
            pltpu.VMEM((K, D), jnp.float32),
            pltpu.VMEM((K, D), jnp.float32),
            pltpu.VMEM_SHARED((N, D), jnp.float32),
            pltpu.SemaphoreType.DMA,
            pltpu.SemaphoreType.DMA,
        ],
    )
    def seg_sum(h_hbm, src_hbm, dst_hbm, out_hbm,
                src_v, dst_v, rows0, rows1, acc_sh, gsem0, gsem1):
        c = lax.axis_index("c")
        s = lax.axis_index("s")
        wid = c * NS + s

        zero16 = jnp.zeros((16,), jnp.float32)

        def zrow(i, carry):
            def zcol(k2, carry2):
                rows0[i, pl.ds(k2 * 16, 16)] = zero16
                return carry2
            return lax.fori_loop(0, D // 16, zcol, carry)

        lax.fori_loop(0, BLK, zrow, 0)

        def zcopy(j, carry):
            b = j * NS + s

            @pl.when(b < NB)
            def _():
                pltpu.sync_copy(rows0, acc_sh.at[pl.ds(b * BLK, BLK)])
            return carry

        lax.fori_loop(0, (NB + NS - 1) // NS, zcopy, 0)

        pltpu.sync_copy(src_hbm.at[wid], src_v)
        pltpu.sync_copy(dst_hbm.at[wid], dst_v)
        plsc.subcore_barrier()

        def start_gather(j, buf, gsem):
            pltpu.async_copy(h_hbm.at[src_v.at[pl.ds(j * K, K)]], buf, gsem)

        def wait_gather(buf, gsem):
            pltpu.make_async_copy(h_hbm.at[src_v.at[pl.ds(0, K)]],
                                  buf, gsem).wait()

        start_gather(0, rows0, gsem0)
        start_gather(1, rows1, gsem1)

        def body(i, carry):
            wait_gather(rows0, gsem0)
            pltpu.sync_copy(rows0, acc_sh.at[dst_v.at[2 * i]], add=True)

            @pl.when(2 * i + 2 < CH)
            def _():
                start_gather(2 * i + 2, rows0, gsem0)

            wait_gather(rows1, gsem1)
            pltpu.sync_copy(rows1, acc_sh.at[dst_v.at[2 * i + 1]], add=True)

            @pl.when(2 * i + 3 < CH)
            def _():
                start_gather(2 * i + 3, rows1, gsem1)
            return carry

        lax.fori_loop(0, CH // 2, body, 0)
        if CH % 2:
            wait_gather(rows0, gsem0)
            pltpu.sync_copy(rows0, acc_sh.at[dst_v.at[CH - 1]], add=True)
        plsc.subcore_barrier()

        def ocopy(j, carry):
            b = j * NS + s

            @pl.when(b < NB)
            def _():
                pltpu.sync_copy(acc_sh.at[pl.ds(b * BLK, BLK)],
                                out_hbm.at[c].at[pl.ds(b * BLK, BLK)])
            return carry

        lax.fori_loop(0, (NB + NS - 1) // NS, ocopy, 0)

    return seg_sum(h, src_r, dst_r)


def _tc_mlp(h, parts, wa, ba, ga, bea, wb, bb, gb, beb):
    def mlp_kernel(h_ref, p_ref, wa_ref, ba_ref, ga_ref, bea_ref,
                   wb_ref, bb_ref, gb_ref, beb_ref, o_ref):
        z = h_ref[...] + p_ref[0] + p_ref[1]
        z = jnp.dot(z, wa_ref[...], preferred_element_type=jnp.float32)
        z = z + ba_ref[...]
        mu = jnp.mean(z, axis=0, keepdims=True)
        var = jnp.mean((z - mu) * (z - mu), axis=0, keepdims=True)
        z = (z - mu) * lax.rsqrt(var + 1e-5) * ga_ref[...] + bea_ref[...]
        z = jnp.maximum(z, 0.0)
        z = jnp.dot(z, wb_ref[...], preferred_element_type=jnp.float32)
        z = z + bb_ref[...]
        mu = jnp.mean(z, axis=0, keepdims=True)
        var = jnp.mean((z - mu) * (z - mu), axis=0, keepdims=True)
        z = (z - mu) * lax.rsqrt(var + 1e-5) * gb_ref[...] + beb_ref[...]
        o_ref[...] = jnp.maximum(z, 0.0)

    return pl.pallas_call(
        mlp_kernel,
        out_shape=jax.ShapeDtypeStruct((N, D), jnp.float32),
    )(h, parts, wa, ba, ga, bea, wb, bb, gb, beb)


PB = 8
NPB = N // PB


def _tc_pool(h3, batch_col, batch8, wl, bl):
    def pool_kernel(h3_ref, b_ref, b8_ref, wl_ref, bl_ref, o_ref,
                    pooled_ref, bm_ref):
        bm_ref[...] = jnp.max(h3_ref[...], axis=1)
        bmin = b8_ref[:, 0:1]
        bmax = b8_ref[:, 7:8]
        bv = b_ref[...]
        bmv = bm_ref[...]
        neg = jnp.float32(-jnp.inf)

        def body(g, carry):
            mpure = (bmin == g) & (bmax == g)
            m0 = jnp.max(jnp.where(mpure, bmv, neg), axis=0)
            s_g = jnp.sum((bv < g).astype(jnp.int32))
            e_g = jnp.sum((bv <= g).astype(jnp.int32))
            blk0 = jnp.minimum(s_g, N - 1) // PB
            blk1 = jnp.maximum(e_g - 1, 0) // PB

            def edge_max(blk):
                rows = h3_ref[pl.ds(blk, 1)][0]
                m = b_ref[pl.ds(blk * PB, PB), :] == g
                return jnp.max(jnp.where(m, rows, neg), axis=0)

            best = jnp.maximum(m0, jnp.maximum(edge_max(blk0), edge_max(blk1)))
            pooled_ref[pl.ds(g, 1), :] = best.reshape(1, D)
            return carry

        lax.fori_loop(0, G, body, 0)
        o_ref[...] = (
            jnp.dot(pooled_ref[...], wl_ref[...],
                    preferred_element_type=jnp.float32)
            + bl_ref[...]
        )

    return pl.pallas_call(
        pool_kernel,
        out_shape=jax.ShapeDtypeStruct((G, OUT), jnp.float32),
        scratch_shapes=[pltpu.VMEM((G, D), jnp.float32),
                        pltpu.VMEM((NPB, D), jnp.float32)],
    )(h3, batch_col, batch8, wl, bl)


def kernel(x, edge_index, batch, Wa, ba, ga, bea, Wb, bb, gb, beb, Wl, bl):
    src_r = edge_index[0].reshape(NW, EPT)
    dst_r = edge_index[1].reshape(NW, CH, K)
    batch_col = batch.reshape(N, 1)
    h = x
    for i in range(L):
        parts = _sc_segment_sum(h, src_r, dst_r)
        h = _tc_mlp(
            h, parts, Wa[i],
            ba[i].reshape(1, D), ga[i].reshape(1, D), bea[i].reshape(1, D),
            Wb[i],
            bb[i].reshape(1, D), gb[i].reshape(1, D), beb[i].reshape(1, D),
        )
    return _tc_pool(h.reshape(NPB, PB, D), batch_col,
                    batch.reshape(NPB, PB), Wl, bl.reshape(1, OUT))


# speedup vs baseline: 1.2323x; 1.2323x over previous
import functools

import jax
import jax.numpy as jnp
from jax import lax
from jax.experimental import pallas as pl
from jax.experimental.pallas import tpu as pltpu
from jax.experimental.pallas import tpu_sc as plsc

N = 10000
E = 320000
D = 128
G = 64
OUT = 10
L = 3

NC = 2
NS = 16
NW = NC * NS
EPT = E // NW
K = 80
CH = EPT // K
BLK = 80
NB = N // BLK


def _sc_segment_sum(h, src_r, dst_r):
    mesh = plsc.VectorSubcoreMesh(core_axis_name="c", subcore_axis_name="s")

    @functools.partial(
        pl.kernel,
        mesh=mesh,
        out_type=jax.ShapeDtypeStruct((NC, N, D), jnp.float32),
        scratch_types=[
            pltpu.VMEM((EPT,), jnp.int32),
            pltpu.VMEM((CH, K), jnp.int32),
            pltpu.VMEM((K, D), jnp.float32),
            pltpu.VMEM((K, D), jnp.float32),
            pltpu.VMEM_SHARED((N, D), jnp.float32),
            pltpu.SemaphoreType.DMA,
            pltpu.SemaphoreType.DMA,
        ],
    )
    def seg_sum(h_hbm, src_hbm, dst_hbm, out_hbm,
                src_v, dst_v, rows0, rows1, acc_sh, gsem0, gsem1):
        c = lax.axis_index("c")
        s = lax.axis_index("s")
        wid = c * NS + s

        zero16 = jnp.zeros((16,), jnp.float32)

        def zrow(i, carry):
            def zcol(k2, carry2):
                rows0[i, pl.ds(k2 * 16, 16)] = zero16
                return carry2
            return lax.fori_loop(0, D // 16, zcol, carry)

        lax.fori_loop(0, BLK, zrow, 0)

        def zcopy(j, carry):
            b = j * NS + s

            @pl.when(b < NB)
            def _():
                pltpu.sync_copy(rows0, acc_sh.at[pl.ds(b * BLK, BLK)])
            return carry

        lax.fori_loop(0, (NB + NS - 1) // NS, zcopy, 0)

        pltpu.sync_copy(src_hbm.at[wid], src_v)
        pltpu.sync_copy(dst_hbm.at[wid], dst_v)
        plsc.subcore_barrier()

        def start_gather(j, buf, gsem):
            pltpu.async_copy(h_hbm.at[src_v.at[pl.ds(j * K, K)]], buf, gsem)

        def wait_gather(buf, gsem):
            pltpu.make_async_copy(h_hbm.at[src_v.at[pl.ds(0, K)]],
                                  buf, gsem).wait()

        start_gather(0, rows0, gsem0)
        start_gather(1, rows1, gsem1)

        def body(i, carry):
            wait_gather(rows0, gsem0)
            pltpu.sync_copy(rows0, acc_sh.at[dst_v.at[2 * i]], add=True)

            @pl.when(2 * i + 2 < CH)
            def _():
                start_gather(2 * i + 2, rows0, gsem0)

            wait_gather(rows1, gsem1)
            pltpu.sync_copy(rows1, acc_sh.at[dst_v.at[2 * i + 1]], add=True)

            @pl.when(2 * i + 3 < CH)
            def _():
                start_gather(2 * i + 3, rows1, gsem1)
            return carry

        lax.fori_loop(0, CH // 2, body, 0)
        if CH % 2:
            wait_gather(rows0, gsem0)
            pltpu.sync_copy(rows0, acc_sh.at[dst_v.at[CH - 1]], add=True)
        plsc.subcore_barrier()

        def ocopy(j, carry):
            b = j * NS + s

            @pl.when(b < NB)
            def _():
                pltpu.sync_copy(acc_sh.at[pl.ds(b * BLK, BLK)],
                                out_hbm.at[c].at[pl.ds(b * BLK, BLK)])
            return carry

        lax.fori_loop(0, (NB + NS - 1) // NS, ocopy, 0)

    return seg_sum(h, src_r, dst_r)


def _tc_mlp(h, parts, wa, ba, ga, bea, wb, bb, gb, beb):
    def mlp_kernel(h_ref, p_ref, wa_ref, ba_ref, ga_ref, bea_ref,
                   wb_ref, bb_ref, gb_ref, beb_ref, o_ref):
        z = h_ref[...] + p_ref[0] + p_ref[1]
        z = jnp.dot(z, wa_ref[...], preferred_element_type=jnp.float32)
        z = z + ba_ref[...]
        mu = jnp.mean(z, axis=0, keepdims=True)
        var = jnp.mean((z - mu) * (z - mu), axis=0, keepdims=True)
        z = (z - mu) * lax.rsqrt(var + 1e-5) * ga_ref[...] + bea_ref[...]
        z = jnp.maximum(z, 0.0)
        z = jnp.dot(z, wb_ref[...], preferred_element_type=jnp.float32)
        z = z + bb_ref[...]
        mu = jnp.mean(z, axis=0, keepdims=True)
        var = jnp.mean((z - mu) * (z - mu), axis=0, keepdims=True)
        z = (z - mu) * lax.rsqrt(var + 1e-5) * gb_ref[...] + beb_ref[...]
        o_ref[...] = jnp.maximum(z, 0.0)

    return pl.pallas_call(
        mlp_kernel,
        out_shape=jax.ShapeDtypeStruct((N, D), jnp.float32),
    )(h, parts, wa, ba, ga, bea, wb, bb, gb, beb)


PB = 8
NPB = N // PB


def _tc_pool(h3, batch_col, batch8, wl, bl):
    def pool_kernel(h3_ref, b_ref, b8_ref, wl_ref, bl_ref, o_ref,
                    pooled_ref, bm_ref, blk0_ref, blk1_ref):
        bm_ref[...] = jnp.max(h3_ref[...], axis=1)
        bmin = b8_ref[:, 0:1]
        bmax = b8_ref[:, 7:8]
        bv = b_ref[...]
        bmv = bm_ref[...]
        neg = jnp.float32(-jnp.inf)

        gid = lax.broadcasted_iota(jnp.int32, (1, G), 1)
        onehot = (bv == gid).astype(jnp.float32)
        counts = jnp.sum(onehot, axis=0, keepdims=True)
        krow = lax.broadcasted_iota(jnp.int32, (G, G), 0)
        gcol = lax.broadcasted_iota(jnp.int32, (G, G), 1)
        lt = (krow < gcol).astype(jnp.float32)
        eye = (krow == gcol).astype(jnp.float32)
        starts = jnp.dot(counts, lt, preferred_element_type=jnp.float32)
        ends = starts + counts
        dn = (((1,), (1,)), ((), ()))
        starts_c = lax.dot_general(eye, starts, dn,
                                   preferred_element_type=jnp.float32)
        ends_c = lax.dot_general(eye, ends, dn,
                                 preferred_element_type=jnp.float32)
        blk0_ref[...] = (
            jnp.minimum(starts_c, N - 1).astype(jnp.int32) // PB)
        blk1_ref[...] = (
            jnp.maximum(ends_c - 1, 0).astype(jnp.int32) // PB)

        def body(g, carry):
            mpure = (bmin == g) & (bmax == g)
            m0 = jnp.max(jnp.where(mpure, bmv, neg), axis=0)
            blk0 = jnp.sum(blk0_ref[pl.ds(g, 1), :])
            blk1 = jnp.sum(blk1_ref[pl.ds(g, 1), :])

            def edge_max(blk):
                rows = h3_ref[pl.ds(blk, 1)][0]
                m = b_ref[pl.ds(blk * PB, PB), :] == g
                return jnp.max(jnp.where(m, rows, neg), axis=0)

            best = jnp.maximum(m0, jnp.maximum(edge_max(blk0), edge_max(blk1)))
            pooled_ref[pl.ds(g, 1), :] = best.reshape(1, D)
            return carry

        lax.fori_loop(0, G, body, 0)
        o_ref[...] = (
            jnp.dot(pooled_ref[...], wl_ref[...],
                    preferred_element_type=jnp.float32)
            + bl_ref[...]
        )

    return pl.pallas_call(
        pool_kernel,
        out_shape=jax.ShapeDtypeStruct((G, OUT), jnp.float32),
        scratch_shapes=[pltpu.VMEM((G, D), jnp.float32),
                        pltpu.VMEM((NPB, D), jnp.float32),
                        pltpu.VMEM((G, 1), jnp.int32),
                        pltpu.VMEM((G, 1), jnp.int32)],
    )(h3, batch_col, batch8, wl, bl)


def kernel(x, edge_index, batch, Wa, ba, ga, bea, Wb, bb, gb, beb, Wl, bl):
    src_r = edge_index[0].reshape(NW, EPT)
    dst_r = edge_index[1].reshape(NW, CH, K)
    batch_col = batch.reshape(N, 1)
    h = x
    for i in range(L):
        parts = _sc_segment_sum(h, src_r, dst_r)
        h = _tc_mlp(
            h, parts, Wa[i],
            ba[i].reshape(1, D), ga[i].reshape(1, D), bea[i].reshape(1, D),
            Wb[i],
            bb[i].reshape(1, D), gb[i].reshape(1, D), beb[i].reshape(1, D),
        )
    return _tc_pool(h.reshape(NPB, PB, D), batch_col,
                    batch.reshape(NPB, PB), Wl, bl.reshape(1, OUT))


# speedup vs baseline: 1.2475x; 1.0124x over previous
import functools

import jax
import jax.numpy as jnp
from jax import lax
from jax.experimental import pallas as pl
from jax.experimental.pallas import tpu as pltpu
from jax.experimental.pallas import tpu_sc as plsc

N = 10000
E = 320000
D = 128
G = 64
OUT = 10
L = 3

NC = 2
NS = 16
NW = NC * NS
EPT = E // NW
K = 80
CH = EPT // K
BLK = 80
NB = N // BLK


def _sc_segment_sum(h, src_r, dst_r):
    mesh = plsc.VectorSubcoreMesh(core_axis_name="c", subcore_axis_name="s")

    @functools.partial(
        pl.kernel,
        mesh=mesh,
        out_type=jax.ShapeDtypeStruct((NC, N, D), jnp.float32),
        scratch_types=[
            pltpu.VMEM((EPT,), jnp.int32),
            pltpu.VMEM((CH, K), jnp.int32),
            pltpu.VMEM((K, D), jnp.float32),
            pltpu.VMEM((K, D), jnp.float32),
            pltpu.VMEM_SHARED((N, D), jnp.float32),
            pltpu.SemaphoreType.DMA,
            pltpu.SemaphoreType.DMA,
        ],
    )
    def seg_sum(h_hbm, src_hbm, dst_hbm, out_hbm,
                src_v, dst_v, rows0, rows1, acc_sh, gsem0, gsem1):
        c = lax.axis_index("c")
        s = lax.axis_index("s")
        wid = c * NS + s

        zero16 = jnp.zeros((16,), jnp.float32)

        def zrow(i, carry):
            def zcol(k2, carry2):
                rows0[i, pl.ds(k2 * 16, 16)] = zero16
                return carry2
            return lax.fori_loop(0, D // 16, zcol, carry)

        lax.fori_loop(0, BLK, zrow, 0)

        def zcopy(j, carry):
            b = j * NS + s

            @pl.when(b < NB)
            def _():
                pltpu.sync_copy(rows0, acc_sh.at[pl.ds(b * BLK, BLK)])
            return carry

        lax.fori_loop(0, (NB + NS - 1) // NS, zcopy, 0)

        pltpu.sync_copy(src_hbm.at[wid], src_v)
        pltpu.sync_copy(dst_hbm.at[wid], dst_v)
        plsc.subcore_barrier()

        def start_gather(j, buf, gsem):
            pltpu.async_copy(h_hbm.at[src_v.at[pl.ds(j * K, K)]], buf, gsem)

        def wait_gather(buf, gsem):
            pltpu.make_async_copy(h_hbm.at[src_v.at[pl.ds(0, K)]],
                                  buf, gsem).wait()

        start_gather(0, rows0, gsem0)
        start_gather(1, rows1, gsem1)

        def body(i, carry):
            wait_gather(rows0, gsem0)
            pltpu.sync_copy(rows0, acc_sh.at[dst_v.at[2 * i]], add=True)

            @pl.when(2 * i + 2 < CH)
            def _():
                start_gather(2 * i + 2, rows0, gsem0)

            wait_gather(rows1, gsem1)
            pltpu.sync_copy(rows1, acc_sh.at[dst_v.at[2 * i + 1]], add=True)

            @pl.when(2 * i + 3 < CH)
            def _():
                start_gather(2 * i + 3, rows1, gsem1)
            return carry

        lax.fori_loop(0, CH // 2, body, 0)
        if CH % 2:
            wait_gather(rows0, gsem0)
            pltpu.sync_copy(rows0, acc_sh.at[dst_v.at[CH - 1]], add=True)
        plsc.subcore_barrier()

        def ocopy(j, carry):
            b = j * NS + s

            @pl.when(b < NB)
            def _():
                pltpu.sync_copy(acc_sh.at[pl.ds(b * BLK, BLK)],
                                out_hbm.at[c].at[pl.ds(b * BLK, BLK)])
            return carry

        lax.fori_loop(0, (NB + NS - 1) // NS, ocopy, 0)

    return seg_sum(h, src_r, dst_r)


def _tc_mlp(h, parts, wa, ba, ga, bea, wb, bb, gb, beb):
    def mlp_kernel(h_ref, p_ref, wa_ref, ba_ref, ga_ref, bea_ref,
                   wb_ref, bb_ref, gb_ref, beb_ref, o_ref):
        z = h_ref[...] + p_ref[0] + p_ref[1]
        z = jnp.dot(z, wa_ref[...], preferred_element_type=jnp.float32)
        z = z + ba_ref[...]
        mu = jnp.mean(z, axis=0, keepdims=True)
        var = jnp.mean((z - mu) * (z - mu), axis=0, keepdims=True)
        z = (z - mu) * lax.rsqrt(var + 1e-5) * ga_ref[...] + bea_ref[...]
        z = jnp.maximum(z, 0.0)
        z = jnp.dot(z, wb_ref[...], preferred_element_type=jnp.float32)
        z = z + bb_ref[...]
        mu = jnp.mean(z, axis=0, keepdims=True)
        var = jnp.mean((z - mu) * (z - mu), axis=0, keepdims=True)
        z = (z - mu) * lax.rsqrt(var + 1e-5) * gb_ref[...] + beb_ref[...]
        o_ref[...] = jnp.maximum(z, 0.0)

    return pl.pallas_call(
        mlp_kernel,
        out_shape=jax.ShapeDtypeStruct((N, D), jnp.float32),
    )(h, parts, wa, ba, ga, bea, wb, bb, gb, beb)


PB = 8
NPB = N // PB


def _tc_mlp_pool(h, parts, wa, ba, ga, bea, wb, bb, gb, beb,
                 batch_col, batch8, wl, bl):
    def mlp_pool_kernel(h_ref, p_ref, wa_ref, ba_ref, ga_ref, bea_ref,
                        wb_ref, bb_ref, gb_ref, beb_ref,
                        b_ref, b8_ref, wl_ref, bl_ref, o_ref,
                        hs_ref, pooled_ref, bm_ref, blk0_ref, blk1_ref):
        z = h_ref[...] + p_ref[0] + p_ref[1]
        z = jnp.dot(z, wa_ref[...], preferred_element_type=jnp.float32)
        z = z + ba_ref[...]
        mu = jnp.mean(z, axis=0, keepdims=True)
        var = jnp.mean((z - mu) * (z - mu), axis=0, keepdims=True)
        z = (z - mu) * lax.rsqrt(var + 1e-5) * ga_ref[...] + bea_ref[...]
        z = jnp.maximum(z, 0.0)
        z = jnp.dot(z, wb_ref[...], preferred_element_type=jnp.float32)
        z = z + bb_ref[...]
        mu = jnp.mean(z, axis=0, keepdims=True)
        var = jnp.mean((z - mu) * (z - mu), axis=0, keepdims=True)
        z = (z - mu) * lax.rsqrt(var + 1e-5) * gb_ref[...] + beb_ref[...]
        z = jnp.maximum(z, 0.0)
        hs_ref[...] = z

        bm_ref[...] = jnp.max(z.reshape(NPB, PB, D), axis=1)
        bmin = b8_ref[:, 0:1]
        bmax = b8_ref[:, 7:8]
        bv = b_ref[...]
        bmv = bm_ref[...]
        neg = jnp.float32(-jnp.inf)

        gid = lax.broadcasted_iota(jnp.int32, (1, G), 1)
        onehot = (bv == gid).astype(jnp.float32)
        counts = jnp.sum(onehot, axis=0, keepdims=True)
        krow = lax.broadcasted_iota(jnp.int32, (G, G), 0)
        gcol = lax.broadcasted_iota(jnp.int32, (G, G), 1)
        lt = (krow < gcol).astype(jnp.float32)
        eye = (krow == gcol).astype(jnp.float32)
        starts = jnp.dot(counts, lt, preferred_element_type=jnp.float32)
        ends = starts + counts
        dn = (((1,), (1,)), ((), ()))
        starts_c = lax.dot_general(eye, starts, dn,
                                   preferred_element_type=jnp.float32)
        ends_c = lax.dot_general(eye, ends, dn,
                                 preferred_element_type=jnp.float32)
        blk0_ref[...] = (
            jnp.minimum(starts_c, N - 1).astype(jnp.int32) // PB)
        blk1_ref[...] = (
            jnp.maximum(ends_c - 1, 0).astype(jnp.int32) // PB)

        def body(g, carry):
            mpure = (bmin == g) & (bmax == g)
            m0 = jnp.max(jnp.where(mpure, bmv, neg), axis=0)
            blk0 = jnp.sum(blk0_ref[pl.ds(g, 1), :])
            blk1 = jnp.sum(blk1_ref[pl.ds(g, 1), :])

            def edge_max(blk):
                rows = hs_ref[pl.ds(blk * PB, PB), :]
                m = b_ref[pl.ds(blk * PB, PB), :] == g
                return jnp.max(jnp.where(m, rows, neg), axis=0)

            best = jnp.maximum(m0, jnp.maximum(edge_max(blk0), edge_max(blk1)))
            pooled_ref[pl.ds(g, 1), :] = best.reshape(1, D)
            return carry

        lax.fori_loop(0, G, body, 0)
        o_ref[...] = (
            jnp.dot(pooled_ref[...], wl_ref[...],
                    preferred_element_type=jnp.float32)
            + bl_ref[...]
        )

    return pl.pallas_call(
        mlp_pool_kernel,
        out_shape=jax.ShapeDtypeStruct((G, OUT), jnp.float32),
        scratch_shapes=[pltpu.VMEM((N, D), jnp.float32),
                        pltpu.VMEM((G, D), jnp.float32),
                        pltpu.VMEM((NPB, D), jnp.float32),
                        pltpu.VMEM((G, 1), jnp.int32),
                        pltpu.VMEM((G, 1), jnp.int32)],
    )(h, parts, wa, ba, ga, bea, wb, bb, gb, beb, batch_col, batch8, wl, bl)


def kernel(x, edge_index, batch, Wa, ba, ga, bea, Wb, bb, gb, beb, Wl, bl):
    src_r = edge_index[0].reshape(NW, EPT)
    dst_r = edge_index[1].reshape(NW, CH, K)
    batch_col = batch.reshape(N, 1)
    h = x
    for i in range(L - 1):
        parts = _sc_segment_sum(h, src_r, dst_r)
        h = _tc_mlp(
            h, parts, Wa[i],
            ba[i].reshape(1, D), ga[i].reshape(1, D), bea[i].reshape(1, D),
            Wb[i],
            bb[i].reshape(1, D), gb[i].reshape(1, D), beb[i].reshape(1, D),
        )
    i = L - 1
    parts = _sc_segment_sum(h, src_r, dst_r)
    return _tc_mlp_pool(
        h, parts, Wa[i],
        ba[i].reshape(1, D), ga[i].reshape(1, D), bea[i].reshape(1, D),
        Wb[i],
        bb[i].reshape(1, D), gb[i].reshape(1, D), beb[i].reshape(1, D),
        batch_col, batch.reshape(NPB, PB), Wl, bl.reshape(1, OUT))
